# baseline (speedup 1.0000x reference)
import functools

import jax
import jax.numpy as jnp
from jax import lax
from jax.experimental import pallas as pl
from jax.experimental.pallas import tpu as pltpu
from jax.experimental.pallas import tpu_sc as plsc

_NC = 2
_NS = 16
_NW = _NC * _NS
_DEGW = 128
_DS = 128


def _make_sc_gather_scatter_add(nsplit, e, n_dst_pad, k):
    e_per_w = e // _NW
    chunks = e_per_w // k
    assert e_per_w * _NW == e and chunks * k == e_per_w and chunks % 2 == 0
    assert k <= 128 and k % 8 == 0
    rpt = n_dst_pad // _NS
    assert rpt * _NS == n_dst_pad and rpt % 8 == 0

    mesh = plsc.VectorSubcoreMesh(core_axis_name="c", subcore_axis_name="s")

    @functools.partial(
        pl.kernel,
        mesh=mesh,
        out_type=(
            [jax.ShapeDtypeStruct((_NC * n_dst_pad, _DS), jnp.float32)] * nsplit
            + [jax.ShapeDtypeStruct((_NC * n_dst_pad, _DEGW), jnp.float32)]
        ),
        scratch_types=(
            [pltpu.VMEM((chunks, k), jnp.int32)] * 2
            + [pltpu.VMEM((k, _DS), jnp.float32)] * (2 * nsplit)
            + [pltpu.VMEM((k, _DEGW), jnp.float32)]
            + [pltpu.VMEM_SHARED((n_dst_pad, _DS), jnp.float32)] * nsplit
            + [pltpu.VMEM_SHARED((n_dst_pad, _DEGW), jnp.float32)]
            + [pltpu.SemaphoreType.DMA] * (4 * nsplit + 2)
        ),
    )
    def kfn(src3_hbm, dst3_hbm, *rest):
        tables = rest[:nsplit]
        zrow_hbm, zdeg_hbm, ones_hbm = rest[nsplit:nsplit + 3]
        sums = rest[nsplit + 3:2 * nsplit + 3]
        deg_hbm = rest[2 * nsplit + 3]
        scr = rest[2 * nsplit + 4:]
        src2d, dst2d = scr[0], scr[1]
        rows = [[scr[2 + 2 * p], scr[3 + 2 * p]] for p in range(nsplit)]
        ones_v = scr[2 + 2 * nsplit]
        accs = scr[3 + 2 * nsplit:3 + 3 * nsplit]
        dacc_sh = scr[3 + 3 * nsplit]
        sems = scr[4 + 3 * nsplit:]
        gsem = [[sems[2 * p], sems[2 * p + 1]] for p in range(nsplit)]
        ssem = [[sems[2 * nsplit + 2 * p], sems[2 * nsplit + 2 * p + 1]]
                for p in range(nsplit)]
        dsem = sems[4 * nsplit:4 * nsplit + 2]

        c = lax.axis_index("c")
        s = lax.axis_index("s")
        wid = s * _NC + c

        pltpu.sync_copy(src3_hbm.at[wid], src2d)
        pltpu.sync_copy(dst3_hbm.at[wid], dst2d)
        pltpu.sync_copy(ones_hbm, ones_v)

        for acc in accs:
            pltpu.sync_copy(zrow_hbm.at[pl.ds(s * rpt, rpt)],
                            acc.at[pl.ds(s * rpt, rpt)])
        pltpu.sync_copy(zdeg_hbm.at[pl.ds(s * rpt, rpt)],
                        dacc_sh.at[pl.ds(s * rpt, rpt)])

        plsc.subcore_barrier()

        for p in range(nsplit):
            pltpu.async_copy(tables[p].at[src2d.at[0]], rows[p][0], gsem[p][0])

        def pair_body(t, carry):
            for b in (0, 1):
                i = 2 * t + b
                for p in range(nsplit):
                    pltpu.make_async_copy(
                        tables[p].at[src2d.at[i]], rows[p][b], gsem[p][b]
                    ).wait()
                @pl.when(i >= 1)
                def _():
                    for p in range(nsplit):
                        pltpu.make_async_copy(
                            rows[p][1 - b], accs[p].at[dst2d.at[i]],
                            ssem[p][1 - b]
                        ).wait()
                @pl.when(i + 1 < chunks)
                def _():
                    for p in range(nsplit):
                        pltpu.async_copy(
                            tables[p].at[src2d.at[i + 1]], rows[p][1 - b],
                            gsem[p][1 - b]
                        )
                for p in range(nsplit):
                    pltpu.async_copy(rows[p][b], accs[p].at[dst2d.at[i]],
                                     ssem[p][b], add=True)
                @pl.when(i >= 2)
                def _():
                    pltpu.make_async_copy(
                        ones_v, dacc_sh.at[dst2d.at[i]], dsem[b]
                    ).wait()
                pltpu.async_copy(ones_v, dacc_sh.at[dst2d.at[i]], dsem[b],
                                 add=True)
            return carry
        lax.fori_loop(0, chunks // 2, pair_body, 0)

        for p in range(nsplit):
            pltpu.make_async_copy(rows[p][1], accs[p].at[dst2d.at[0]],
                                  ssem[p][1]).wait()
        for b in (0, 1):
            pltpu.make_async_copy(ones_v, dacc_sh.at[dst2d.at[0]],
                                  dsem[b]).wait()

        plsc.subcore_barrier()

        row0 = c * n_dst_pad + s * rpt
        for acc, sum_hbm in zip(accs, sums):
            pltpu.sync_copy(acc.at[pl.ds(s * rpt, rpt)],
                            sum_hbm.at[pl.ds(row0, rpt)])
        pltpu.sync_copy(dacc_sh.at[pl.ds(s * rpt, rpt)],
                        deg_hbm.at[pl.ds(row0, rpt)])

    return kfn


_K1, _K2 = 32, 32
_E1P = 327680
_E2P = 32768
_L1CH = _E1P // _NW // _K1
_L2CH = _E2P // _NW // _K2
_sc_layer1 = _make_sc_gather_scatter_add(nsplit=1, e=_E1P, n_dst_pad=2048, k=_K1)
_sc_layer2 = _make_sc_gather_scatter_add(nsplit=2, e=_E2P, n_dst_pad=512, k=_K2)


def _tc_layer(n, n_pad, h_dst, psums, pdeg, w_self, w_neigh, b):
    d_out = w_self.shape[1]
    nsplit = len(psums)

    def body(*refs):
        hd_ref = refs[0]
        p_refs = refs[1:1 + nsplit]
        dg_ref, ws_ref, wn_ref, b_ref, o_ref = refs[1 + nsplit:]
        hd = hd_ref[...]
        sums = jnp.concatenate(
            [p[...][0:n] + p[...][n_pad:n_pad + n] for p in p_refs], axis=1)
        degs = dg_ref[...][:, 0:1]
        deg = degs[0:n] + degs[n_pad:n_pad + n]
        neigh = sums / jnp.maximum(deg, 1.0)
        z = (jnp.dot(hd, ws_ref[...], preferred_element_type=jnp.float32,
                     precision=lax.Precision.HIGHEST)
             + jnp.dot(neigh, wn_ref[...], preferred_element_type=jnp.float32,
                       precision=lax.Precision.HIGHEST)
             + b_ref[...])
        scale = jnp.mean(jnp.sqrt(jnp.sum(hd * hd, axis=1, keepdims=True)))
        inv = lax.rsqrt(jnp.sum(z * z, axis=1, keepdims=True))
        zn = z * (scale * inv)
        o_ref[...] = jnp.where(zn >= 0, zn, zn * 0.01)

    return pl.pallas_call(
        body,
        out_shape=jax.ShapeDtypeStruct((n, d_out), jnp.float32),
    )(h_dst, *psums, pdeg, w_self, w_neigh, b.reshape(1, d_out))


def _pad_edges(src, dst, e_pad, dummy_dst, ch, k):
    npad = e_pad - src.shape[0]
    if npad:
        src = jnp.concatenate([src, jnp.zeros((npad,), jnp.int32)])
        dst = jnp.concatenate([dst, jnp.full((npad,), dummy_dst, jnp.int32)])
    return src.reshape(_NW, ch, k), dst.reshape(_NW, ch, k)


def kernel(x, edge_src1, edge_dst1, edge_src2, edge_dst2, num_dst1, num_dst2,
           W_self1, W_neigh1, b1, W_self2, W_neigh2, b2):
    N1, N2 = 2000, 500
    NP1, NP2 = 2048, 512

    h_dst1 = lax.dynamic_slice_in_dim(x, num_dst1 - N1, N1, axis=0)
    zrow1 = jnp.zeros((NP1, _DS), jnp.float32)
    zdeg1 = jnp.zeros((NP1, _DEGW), jnp.float32)
    ones1 = jnp.ones((_K1, _DEGW), jnp.float32)
    src1_3, dst1_3 = _pad_edges(edge_src1, edge_dst1, _E1P, NP1 - 1, _L1CH, _K1)
    sum1, deg1 = _sc_layer1(src1_3, dst1_3, x, zrow1, zdeg1, ones1)
    h1 = _tc_layer(N1, NP1, h_dst1, [sum1], deg1, W_self1, W_neigh1, b1)

    h_dst2 = lax.dynamic_slice_in_dim(h1, num_dst2 - N2, N2, axis=0)
    zrow2 = jnp.zeros((NP2, _DS), jnp.float32)
    zdeg2 = jnp.zeros((NP2, _DEGW), jnp.float32)
    ones2 = jnp.ones((_K2, _DEGW), jnp.float32)
    src2_3, dst2_3 = _pad_edges(edge_src2, edge_dst2, _E2P, NP2 - 1, _L2CH, _K2)
    h1_lo = h1[:, :_DS]
    h1_hi = h1[:, _DS:]
    sum2a, sum2b, deg2 = _sc_layer2(src2_3, dst2_3, h1_lo, h1_hi,
                                    zrow2, zdeg2, ones2)
    out = _tc_layer(N2, NP2, h_dst2, [sum2a, sum2b], deg2,
                    W_self2, W_neigh2, b2)
    return out


# speedup vs baseline: 1.0816x; 1.0816x over previous
import functools

import jax
import jax.numpy as jnp
from jax import lax
from jax.experimental import pallas as pl
from jax.experimental.pallas import tpu as pltpu
from jax.experimental.pallas import tpu_sc as plsc

_NC = 2
_NS = 16
_NW = _NC * _NS
_DEGW = 128
_DS = 128


def _make_sc_gather_scatter_add(nsplit, e, n_dst_pad, k):
    e_per_w = e // _NW
    chunks = e_per_w // k
    assert e_per_w * _NW == e and chunks * k == e_per_w and chunks % 2 == 0
    assert k <= 128 and k % 8 == 0
    rpt = n_dst_pad // _NS
    assert rpt * _NS == n_dst_pad and rpt % 8 == 0

    mesh = plsc.VectorSubcoreMesh(core_axis_name="c", subcore_axis_name="s")

    @functools.partial(
        pl.kernel,
        mesh=mesh,
        out_type=(
            [jax.ShapeDtypeStruct((_NC * n_dst_pad, _DS), jnp.float32)] * nsplit
            + [jax.ShapeDtypeStruct((_NC * n_dst_pad, _DEGW), jnp.float32)]
        ),
        scratch_types=(
            [pltpu.VMEM((chunks, k), jnp.int32)] * 2
            + [pltpu.VMEM((k, _DS), jnp.float32)] * (2 * nsplit)
            + [pltpu.VMEM((k, _DEGW), jnp.float32)]
            + [pltpu.VMEM_SHARED((n_dst_pad, _DS), jnp.float32)] * nsplit
            + [pltpu.VMEM_SHARED((n_dst_pad, _DEGW), jnp.float32)]
            + [pltpu.SemaphoreType.DMA] * (4 * nsplit + 2)
        ),
    )
    def kfn(src3_hbm, dst3_hbm, *rest):
        tables = rest[:nsplit]
        zrow_hbm, zdeg_hbm, ones_hbm = rest[nsplit:nsplit + 3]
        sums = rest[nsplit + 3:2 * nsplit + 3]
        deg_hbm = rest[2 * nsplit + 3]
        scr = rest[2 * nsplit + 4:]
        src2d, dst2d = scr[0], scr[1]
        rows = [[scr[2 + 2 * p], scr[3 + 2 * p]] for p in range(nsplit)]
        ones_v = scr[2 + 2 * nsplit]
        accs = scr[3 + 2 * nsplit:3 + 3 * nsplit]
        dacc_sh = scr[3 + 3 * nsplit]
        sems = scr[4 + 3 * nsplit:]
        gsem = [[sems[2 * p], sems[2 * p + 1]] for p in range(nsplit)]
        ssem = [[sems[2 * nsplit + 2 * p], sems[2 * nsplit + 2 * p + 1]]
                for p in range(nsplit)]
        dsem = sems[4 * nsplit:4 * nsplit + 2]

        c = lax.axis_index("c")
        s = lax.axis_index("s")
        wid = s * _NC + c

        pltpu.sync_copy(src3_hbm.at[wid], src2d)
        pltpu.sync_copy(dst3_hbm.at[wid], dst2d)
        pltpu.sync_copy(ones_hbm, ones_v)

        for acc in accs:
            pltpu.sync_copy(zrow_hbm.at[pl.ds(s * rpt, rpt)],
                            acc.at[pl.ds(s * rpt, rpt)])
        pltpu.sync_copy(zdeg_hbm.at[pl.ds(s * rpt, rpt)],
                        dacc_sh.at[pl.ds(s * rpt, rpt)])

        plsc.subcore_barrier()

        for p in range(nsplit):
            pltpu.async_copy(tables[p].at[src2d.at[0]], rows[p][0], gsem[p][0])

        def pair_body(t, carry):
            for b in (0, 1):
                i = 2 * t + b
                for p in range(nsplit):
                    pltpu.make_async_copy(
                        tables[p].at[src2d.at[i]], rows[p][b], gsem[p][b]
                    ).wait()
                @pl.when(i >= 1)
                def _():
                    for p in range(nsplit):
                        pltpu.make_async_copy(
                            rows[p][1 - b], accs[p].at[dst2d.at[i]],
                            ssem[p][1 - b]
                        ).wait()
                @pl.when(i + 1 < chunks)
                def _():
                    for p in range(nsplit):
                        pltpu.async_copy(
                            tables[p].at[src2d.at[i + 1]], rows[p][1 - b],
                            gsem[p][1 - b]
                        )
                for p in range(nsplit):
                    pltpu.async_copy(rows[p][b], accs[p].at[dst2d.at[i]],
                                     ssem[p][b], add=True)
                @pl.when(i >= 2)
                def _():
                    pltpu.make_async_copy(
                        ones_v, dacc_sh.at[dst2d.at[i]], dsem[b]
                    ).wait()
                pltpu.async_copy(ones_v, dacc_sh.at[dst2d.at[i]], dsem[b],
                                 add=True)
            return carry
        lax.fori_loop(0, chunks // 2, pair_body, 0)

        for p in range(nsplit):
            pltpu.make_async_copy(rows[p][1], accs[p].at[dst2d.at[0]],
                                  ssem[p][1]).wait()
        for b in (0, 1):
            pltpu.make_async_copy(ones_v, dacc_sh.at[dst2d.at[0]],
                                  dsem[b]).wait()

        plsc.subcore_barrier()

        row0 = c * n_dst_pad + s * rpt
        for acc, sum_hbm in zip(accs, sums):
            pltpu.sync_copy(acc.at[pl.ds(s * rpt, rpt)],
                            sum_hbm.at[pl.ds(row0, rpt)])
        pltpu.sync_copy(dacc_sh.at[pl.ds(s * rpt, rpt)],
                        deg_hbm.at[pl.ds(row0, rpt)])

    return kfn


_K1, _K2 = 40, 40
_E1P = 320000
_E2P = 40960
_L1CH = _E1P // _NW // _K1
_L2CH = _E2P // _NW // _K2
_sc_layer1 = _make_sc_gather_scatter_add(nsplit=1, e=_E1P, n_dst_pad=2048, k=_K1)
_sc_layer2 = _make_sc_gather_scatter_add(nsplit=2, e=_E2P, n_dst_pad=512, k=_K2)


def _tc_layer(n, n_pad, h_dst, psums, pdeg, w_self, w_neigh, b):
    d_out = w_self.shape[1]
    nsplit = len(psums)

    def body(*refs):
        hd_ref = refs[0]
        p_refs = refs[1:1 + nsplit]
        dg_ref, ws_ref, wn_ref, b_ref, o_ref = refs[1 + nsplit:]
        hd = hd_ref[...]
        sums = jnp.concatenate(
            [p[...][0:n] + p[...][n_pad:n_pad + n] for p in p_refs], axis=1)
        degs = dg_ref[...][:, 0:1]
        deg = degs[0:n] + degs[n_pad:n_pad + n]
        neigh = sums / jnp.maximum(deg, 1.0)
        z = (jnp.dot(hd, ws_ref[...], preferred_element_type=jnp.float32,
                     precision=lax.Precision.HIGHEST)
             + jnp.dot(neigh, wn_ref[...], preferred_element_type=jnp.float32,
                       precision=lax.Precision.HIGHEST)
             + b_ref[...])
        scale = jnp.mean(jnp.sqrt(jnp.sum(hd * hd, axis=1, keepdims=True)))
        inv = lax.rsqrt(jnp.sum(z * z, axis=1, keepdims=True))
        zn = z * (scale * inv)
        o_ref[...] = jnp.where(zn >= 0, zn, zn * 0.01)

    return pl.pallas_call(
        body,
        out_shape=jax.ShapeDtypeStruct((n, d_out), jnp.float32),
    )(h_dst, *psums, pdeg, w_self, w_neigh, b.reshape(1, d_out))


def _pad_edges(src, dst, e_pad, dummy_lo, nspread, ch, k):
    npad = e_pad - src.shape[0]
    if npad:
        src = jnp.concatenate([src, jnp.zeros((npad,), jnp.int32)])
        spread = dummy_lo + (jnp.arange(npad, dtype=jnp.int32) % nspread)
        dst = jnp.concatenate([dst, spread])
    return src.reshape(_NW, ch, k), dst.reshape(_NW, ch, k)


def kernel(x, edge_src1, edge_dst1, edge_src2, edge_dst2, num_dst1, num_dst2,
           W_self1, W_neigh1, b1, W_self2, W_neigh2, b2):
    N1, N2 = 2000, 500
    NP1, NP2 = 2048, 512

    h_dst1 = lax.dynamic_slice_in_dim(x, num_dst1 - N1, N1, axis=0)
    zrow1 = jnp.zeros((NP1, _DS), jnp.float32)
    zdeg1 = jnp.zeros((NP1, _DEGW), jnp.float32)
    ones1 = jnp.ones((_K1, _DEGW), jnp.float32)
    src1_3, dst1_3 = _pad_edges(edge_src1, edge_dst1, _E1P, N1, NP1 - N1, _L1CH, _K1)
    sum1, deg1 = _sc_layer1(src1_3, dst1_3, x, zrow1, zdeg1, ones1)
    h1 = _tc_layer(N1, NP1, h_dst1, [sum1], deg1, W_self1, W_neigh1, b1)

    h_dst2 = lax.dynamic_slice_in_dim(h1, num_dst2 - N2, N2, axis=0)
    zrow2 = jnp.zeros((NP2, _DS), jnp.float32)
    zdeg2 = jnp.zeros((NP2, _DEGW), jnp.float32)
    ones2 = jnp.ones((_K2, _DEGW), jnp.float32)
    src2_3, dst2_3 = _pad_edges(edge_src2, edge_dst2, _E2P, N2, NP2 - N2, _L2CH, _K2)
    h1_lo = h1[:, :_DS]
    h1_hi = h1[:, _DS:]
    sum2a, sum2b, deg2 = _sc_layer2(src2_3, dst2_3, h1_lo, h1_hi,
                                    zrow2, zdeg2, ones2)
    out = _tc_layer(N2, NP2, h_dst2, [sum2a, sum2b], deg2,
                    W_self2, W_neigh2, b2)
    return out


# speedup vs baseline: 2.0622x; 1.9066x over previous
import functools

import jax
import jax.numpy as jnp
from jax import lax
from jax.experimental import pallas as pl
from jax.experimental.pallas import tpu as pltpu
from jax.experimental.pallas import tpu_sc as plsc

_NC = 2
_NS = 16
_NW = _NC * _NS
_DEGW = 128
_DS = 128


def _make_sc_gather_scatter_add(nsplit, e, n_dst_pad, k):
    e_per_w = e // _NW
    chunks = e_per_w // k
    assert e_per_w * _NW == e and chunks * k == e_per_w and chunks % 2 == 0
    assert k <= 128 and k % 8 == 0
    rpt = n_dst_pad // _NS
    assert rpt * _NS == n_dst_pad and rpt % 8 == 0

    mesh = plsc.VectorSubcoreMesh(core_axis_name="c", subcore_axis_name="s")

    @functools.partial(
        pl.kernel,
        mesh=mesh,
        out_type=(
            [jax.ShapeDtypeStruct((_NC * n_dst_pad, _DS), jnp.float32)] * nsplit
            + [jax.ShapeDtypeStruct((_NC * n_dst_pad, _DEGW), jnp.float32)]
        ),
        scratch_types=(
            [pltpu.VMEM((chunks, k), jnp.int32)] * 2
            + [pltpu.VMEM((k, _DS), jnp.float32)] * (2 * nsplit)
            + [pltpu.VMEM((k, _DEGW), jnp.float32)]
            + [pltpu.VMEM_SHARED((n_dst_pad, _DS), jnp.float32)] * nsplit
            + [pltpu.VMEM_SHARED((n_dst_pad, _DEGW), jnp.float32)]
            + [pltpu.SemaphoreType.DMA] * (4 * nsplit + 2)
        ),
    )
    def kfn(src3_hbm, dst3_hbm, *rest):
        tables = rest[:nsplit]
        zrow_hbm, zdeg_hbm, ones_hbm = rest[nsplit:nsplit + 3]
        sums = rest[nsplit + 3:2 * nsplit + 3]
        deg_hbm = rest[2 * nsplit + 3]
        scr = rest[2 * nsplit + 4:]
        src2d, dst2d = scr[0], scr[1]
        rows = [[scr[2 + 2 * p], scr[3 + 2 * p]] for p in range(nsplit)]
        ones_v = scr[2 + 2 * nsplit]
        accs = scr[3 + 2 * nsplit:3 + 3 * nsplit]
        dacc_sh = scr[3 + 3 * nsplit]
        sems = scr[4 + 3 * nsplit:]
        gsem = [[sems[2 * p], sems[2 * p + 1]] for p in range(nsplit)]
        ssem = [[sems[2 * nsplit + 2 * p], sems[2 * nsplit + 2 * p + 1]]
                for p in range(nsplit)]
        dsem = sems[4 * nsplit:4 * nsplit + 2]

        c = lax.axis_index("c")
        s = lax.axis_index("s")
        wid = s * _NC + c

        pltpu.sync_copy(src3_hbm.at[wid], src2d)
        pltpu.sync_copy(dst3_hbm.at[wid], dst2d)
        pltpu.sync_copy(ones_hbm, ones_v)

        for acc in accs:
            pltpu.sync_copy(zrow_hbm.at[pl.ds(s * rpt, rpt)],
                            acc.at[pl.ds(s * rpt, rpt)])
        pltpu.sync_copy(zdeg_hbm.at[pl.ds(s * rpt, rpt)],
                        dacc_sh.at[pl.ds(s * rpt, rpt)])

        plsc.subcore_barrier()

        for p in range(nsplit):
            pltpu.async_copy(tables[p].at[src2d.at[0]], rows[p][0], gsem[p][0])

        def pair_body(t, carry):
            for b in (0, 1):
                i = 2 * t + b
                for p in range(nsplit):
                    pltpu.make_async_copy(
                        tables[p].at[src2d.at[i]], rows[p][b], gsem[p][b]
                    ).wait()
                @pl.when(i >= 1)
                def _():
                    for p in range(nsplit):
                        pltpu.make_async_copy(
                            rows[p][1 - b], accs[p].at[dst2d.at[i]],
                            ssem[p][1 - b]
                        ).wait()
                @pl.when(i + 1 < chunks)
                def _():
                    for p in range(nsplit):
                        pltpu.async_copy(
                            tables[p].at[src2d.at[i + 1]], rows[p][1 - b],
                            gsem[p][1 - b]
                        )
                for p in range(nsplit):
                    pltpu.async_copy(rows[p][b], accs[p].at[dst2d.at[i]],
                                     ssem[p][b], add=True)
                @pl.when(i >= 2)
                def _():
                    pltpu.make_async_copy(
                        ones_v, dacc_sh.at[dst2d.at[i]], dsem[b]
                    ).wait()
                pltpu.async_copy(ones_v, dacc_sh.at[dst2d.at[i]], dsem[b],
                                 add=True)
            return carry
        lax.fori_loop(0, chunks // 2, pair_body, 0)

        for p in range(nsplit):
            pltpu.make_async_copy(rows[p][1], accs[p].at[dst2d.at[0]],
                                  ssem[p][1]).wait()
        for b in (0, 1):
            pltpu.make_async_copy(ones_v, dacc_sh.at[dst2d.at[0]],
                                  dsem[b]).wait()

        plsc.subcore_barrier()

        row0 = c * n_dst_pad + s * rpt
        for acc, sum_hbm in zip(accs, sums):
            pltpu.sync_copy(acc.at[pl.ds(s * rpt, rpt)],
                            sum_hbm.at[pl.ds(row0, rpt)])
        pltpu.sync_copy(dacc_sh.at[pl.ds(s * rpt, rpt)],
                        deg_hbm.at[pl.ds(row0, rpt)])

    return kfn


_K1, _K2 = 40, 64
_E1P = 320000
_E2P = 32768
_L1CH = _E1P // _NW // _K1
_L2CH = _E2P // _NW // _K2
_sc_layer1 = _make_sc_gather_scatter_add(nsplit=1, e=_E1P, n_dst_pad=2048, k=_K1)
_sc_layer2 = _make_sc_gather_scatter_add(nsplit=2, e=_E2P, n_dst_pad=512, k=_K2)


def _tc_layer(n, n_pad, h_dst, psums, pdeg, w_self, w_neigh, b):
    d_out = w_self.shape[1]
    nsplit = len(psums)

    def body(*refs):
        hd_ref = refs[0]
        p_refs = refs[1:1 + nsplit]
        dg_ref, ws_ref, wn_ref, b_ref, o_ref = refs[1 + nsplit:]
        hd = hd_ref[...]
        sums = jnp.concatenate(
            [p[...][0:n] + p[...][n_pad:n_pad + n] for p in p_refs], axis=1)
        degs = dg_ref[...][:, 0:1]
        deg = degs[0:n] + degs[n_pad:n_pad + n]
        neigh = sums / jnp.maximum(deg, 1.0)
        z = (jnp.dot(hd, ws_ref[...], preferred_element_type=jnp.float32,
                     precision=lax.Precision.HIGHEST)
             + jnp.dot(neigh, wn_ref[...], preferred_element_type=jnp.float32,
                       precision=lax.Precision.HIGHEST)
             + b_ref[...])
        scale = jnp.mean(jnp.sqrt(jnp.sum(hd * hd, axis=1, keepdims=True)))
        inv = lax.rsqrt(jnp.sum(z * z, axis=1, keepdims=True))
        zn = z * (scale * inv)
        o_ref[...] = jnp.where(zn >= 0, zn, zn * 0.01)

    return pl.pallas_call(
        body,
        out_shape=jax.ShapeDtypeStruct((n, d_out), jnp.float32),
    )(h_dst, *psums, pdeg, w_self, w_neigh, b.reshape(1, d_out))


def _pad_edges(src, dst, e_pad, dummy_dst, ch, k):
    npad = e_pad - src.shape[0]
    if npad:
        src = jnp.concatenate([src, jnp.zeros((npad,), jnp.int32)])
        dst = jnp.concatenate([dst, jnp.full((npad,), dummy_dst, jnp.int32)])
    return src.reshape(_NW, ch, k), dst.reshape(_NW, ch, k)


def kernel(x, edge_src1, edge_dst1, edge_src2, edge_dst2, num_dst1, num_dst2,
           W_self1, W_neigh1, b1, W_self2, W_neigh2, b2):
    N1, N2 = 2000, 500
    NP1, NP2 = 2048, 512

    h_dst1 = lax.dynamic_slice_in_dim(x, num_dst1 - N1, N1, axis=0)
    zrow1 = jnp.zeros((NP1, _DS), jnp.float32)
    zdeg1 = jnp.zeros((NP1, _DEGW), jnp.float32)
    ones1 = jnp.ones((_K1, _DEGW), jnp.float32)
    src1_3, dst1_3 = _pad_edges(edge_src1, edge_dst1, _E1P, NP1 - 1, _L1CH, _K1)
    sum1, deg1 = _sc_layer1(src1_3, dst1_3, x, zrow1, zdeg1, ones1)
    h1 = _tc_layer(N1, NP1, h_dst1, [sum1], deg1, W_self1, W_neigh1, b1)

    h_dst2 = lax.dynamic_slice_in_dim(h1, num_dst2 - N2, N2, axis=0)
    zrow2 = jnp.zeros((NP2, _DS), jnp.float32)
    zdeg2 = jnp.zeros((NP2, _DEGW), jnp.float32)
    ones2 = jnp.ones((_K2, _DEGW), jnp.float32)
    src2_3, dst2_3 = _pad_edges(edge_src2, edge_dst2, _E2P, NP2 - 1, _L2CH, _K2)
    h1_lo = h1[:, :_DS]
    h1_hi = h1[:, _DS:]
    sum2a, sum2b, deg2 = _sc_layer2(src2_3, dst2_3, h1_lo, h1_hi,
                                    zrow2, zdeg2, ones2)
    out = _tc_layer(N2, NP2, h_dst2, [sum2a, sum2b], deg2,
                    W_self2, W_neigh2, b2)
    return out
